# baseline (speedup 1.0000x reference)
import functools

import jax
import jax.numpy as jnp
from jax.experimental import pallas as pl
from jax.experimental.pallas import tpu as pltpu

B, N, T, C_IN, C_SP, C_OUT = 8, 1024, 16, 32, 16, 32
T1 = T - 2
T2 = T1 - 2
BN_COUNT = B * T2 * C_OUT
EPS = 1e-5


def _conv_weight_2d_t(w, t_in, t_out):
    c = w.shape[1]
    band = jnp.concatenate([w[:, :, 0, 0], w[:, :, 0, 1], w[:, :, 0, 2]], axis=1)
    band = jnp.pad(band, ((0, 0), (0, (t_in - 3) * c)))
    rows = [jnp.roll(band, c * p, axis=1) for p in range(t_out)]
    return jnp.concatenate(rows, axis=0)


def _theta_blockdiag_t(theta, t_len):
    c, s = theta.shape
    band = jnp.pad(theta.T, ((0, 0), (0, (t_len - 1) * c)))
    rows = [jnp.roll(band, c * q, axis=1) for q in range(t_len)]
    return jnp.concatenate(rows, axis=0)


def _stgcn_body(x_ref, at_ref, wc_ref, b13_ref, b2_ref,
                th_ref, vc_ref, c13_ref, c2_ref,
                g_ref, be_ref, out_ref, t3_ref, s1_ref, s2_ref):
    i = pl.program_id(0)
    H1 = T1 * C_OUT
    H2 = T2 * C_OUT

    @pl.when(i < B)
    def _compute():
        x = x_ref[0].astype(jnp.bfloat16)

        zc = jnp.dot(wc_ref[...], x, preferred_element_type=jnp.float32)
        z13 = zc[:H1] + b13_ref[...]
        z2 = zc[H1:] + b2_ref[...]
        sig = 1.0 / (1.0 + jnp.exp(-z2))
        t_feat = jnp.maximum(z13 + sig, 0.0).astype(jnp.bfloat16)

        u = jnp.dot(th_ref[...], t_feat, preferred_element_type=jnp.float32)
        m = jax.lax.dot_general(u.astype(jnp.bfloat16), at_ref[...],
                                (((1,), (1,)), ((), ())),
                                preferred_element_type=jnp.float32)
        t2 = jnp.maximum(m, 0.0).astype(jnp.bfloat16)

        yc = jnp.dot(vc_ref[...], t2, preferred_element_type=jnp.float32)
        y13 = yc[:H2] + c13_ref[...]
        y2 = yc[H2:] + c2_ref[...]
        sig2 = 1.0 / (1.0 + jnp.exp(-y2))
        t3 = jnp.maximum(y13 + sig2, 0.0)

        t3_ref[i] = t3
        rs = jnp.sum(t3, axis=0, keepdims=True)
        rq = jnp.sum(t3 * t3, axis=0, keepdims=True)

        @pl.when(i == 0)
        def _():
            s1_ref[...] = rs
            s2_ref[...] = rq

        @pl.when(i > 0)
        def _():
            s1_ref[...] = s1_ref[...] + rs
            s2_ref[...] = s2_ref[...] + rq

    @pl.when(i >= B)
    def _normalize():
        bb = i - B
        inv_n = 1.0 / BN_COUNT
        mean = s1_ref[...] * inv_n
        var = s2_ref[...] * inv_n - mean * mean
        scale = g_ref[...] * jax.lax.rsqrt(var + EPS)
        shift = be_ref[...] - mean * scale
        out_ref[0] = t3_ref[bb] * scale + shift


@functools.partial(jax.jit, static_argnames=())
def kernel(X, A_hat, t1_w1, t1_b1, t1_w2, t1_b2, t1_w3, t1_b3, Theta1,
           t2_w1, t2_b1, t2_w2, t2_b2, t2_w3, t2_b3, bn_gamma, bn_beta):
    bf = jnp.bfloat16
    wc = jnp.concatenate([
        _conv_weight_2d_t(t1_w1, T, T1) + _conv_weight_2d_t(t1_w3, T, T1),
        _conv_weight_2d_t(t1_w2, T, T1)], axis=0).astype(bf)
    b13 = jnp.tile(t1_b1 + t1_b3, T1)[:, None]
    b2 = jnp.tile(t1_b2, T1)[:, None]
    th = _theta_blockdiag_t(Theta1, T1).astype(bf)
    vc = jnp.concatenate([
        _conv_weight_2d_t(t2_w1, T1, T2) + _conv_weight_2d_t(t2_w3, T1, T2),
        _conv_weight_2d_t(t2_w2, T1, T2)], axis=0).astype(bf)
    c13 = jnp.tile(t2_b1 + t2_b3, T2)[:, None]
    c2 = jnp.tile(t2_b2, T2)[:, None]
    xt = jnp.transpose(X, (0, 2, 3, 1)).reshape(B, T * C_IN, N)
    at = A_hat.astype(bf)
    g = bn_gamma[None, :]
    be = bn_beta[None, :]

    full = lambda shape: pl.BlockSpec(shape, lambda i: (0,) * len(shape))
    out = pl.pallas_call(
        _stgcn_body,
        grid=(2 * B,),
        in_specs=[
            pl.BlockSpec((1, T * C_IN, N),
                         lambda i: (jnp.minimum(i, B - 1), 0, 0)),
            full((N, N)),
            full((2 * T1 * C_OUT, T * C_IN)),
            full((T1 * C_OUT, 1)),
            full((T1 * C_OUT, 1)),
            full((T1 * C_SP, T1 * C_OUT)),
            full((2 * T2 * C_OUT, T1 * C_SP)),
            full((T2 * C_OUT, 1)),
            full((T2 * C_OUT, 1)),
            full((1, N)),
            full((1, N)),
        ],
        out_specs=pl.BlockSpec((1, T2 * C_OUT, N),
                               lambda i: (jnp.maximum(i - B, 0), 0, 0)),
        out_shape=jax.ShapeDtypeStruct((B, T2 * C_OUT, N), jnp.float32),
        scratch_shapes=[
            pltpu.VMEM((B, T2 * C_OUT, N), jnp.float32),
            pltpu.VMEM((1, N), jnp.float32),
            pltpu.VMEM((1, N), jnp.float32),
        ],
    )(xt, at, wc, b13, b2, th, vc, c13, c2, g, be)
    return jnp.transpose(out.reshape(B, T2, C_OUT, N), (0, 3, 1, 2))


# speedup vs baseline: 1.9028x; 1.9028x over previous
import functools

import jax
import jax.numpy as jnp
from jax.experimental import pallas as pl
from jax.experimental.pallas import tpu as pltpu

B, N, T, C_IN, C_SP, C_OUT = 8, 1024, 16, 32, 16, 32
T1 = T - 2
T2 = T1 - 2
H1 = T1 * C_OUT
H2 = T2 * C_OUT
BN_COUNT = B * T2 * C_OUT
EPS = 1e-5


def _stgcn_body(x_ref, a_ref, band13_ref, band2_ref, thsm_ref,
                vband13_ref, vband2_ref, b13_ref, b2_ref, c13_ref, c2_ref,
                g_ref, be_ref, out_ref,
                t3_ref, s1_ref, s2_ref, at_s, wc_s, th_s, vc_s):
    i = pl.program_id(0)

    @pl.when(i == 0)
    def _build_weights():
        at_s[...] = a_ref[...].astype(jnp.bfloat16)
        wc_s[...] = jnp.zeros((2 * H1, T * C_IN), jnp.bfloat16)
        th_s[...] = jnp.zeros((T1 * C_SP, H1), jnp.bfloat16)
        vc_s[...] = jnp.zeros((2 * H2, T1 * C_SP), jnp.bfloat16)
        for p in range(T1):
            wc_s[32 * p:32 * p + 32, 32 * p:32 * p + 96] = band13_ref[...]
            wc_s[H1 + 32 * p:H1 + 32 * p + 32, 32 * p:32 * p + 96] = band2_ref[...]
            th_s[16 * p:16 * p + 16, 32 * p:32 * p + 32] = thsm_ref[...]
        for p in range(T2):
            vc_s[32 * p:32 * p + 32, 16 * p:16 * p + 48] = vband13_ref[...]
            vc_s[H2 + 32 * p:H2 + 32 * p + 32, 16 * p:16 * p + 48] = vband2_ref[...]

    @pl.when(i < B)
    def _compute():
        x = x_ref[0].astype(jnp.bfloat16)

        zc = jnp.dot(wc_s[...], x, preferred_element_type=jnp.float32)
        z13 = zc[:H1] + b13_ref[...]
        z2 = zc[H1:] + b2_ref[...]
        sig = 1.0 / (1.0 + jnp.exp(-z2))
        t_feat = jnp.maximum(z13 + sig, 0.0).astype(jnp.bfloat16)

        u = jnp.dot(th_s[...], t_feat, preferred_element_type=jnp.float32)
        m = jax.lax.dot_general(u.astype(jnp.bfloat16), at_s[...],
                                (((1,), (1,)), ((), ())),
                                preferred_element_type=jnp.float32)
        t2 = jnp.maximum(m, 0.0).astype(jnp.bfloat16)

        yc = jnp.dot(vc_s[...], t2, preferred_element_type=jnp.float32)
        y13 = yc[:H2] + c13_ref[...]
        y2 = yc[H2:] + c2_ref[...]
        sig2 = 1.0 / (1.0 + jnp.exp(-y2))
        t3 = jnp.maximum(y13 + sig2, 0.0)

        t3_ref[i] = t3
        rs = jnp.sum(t3, axis=0, keepdims=True)
        rq = jnp.sum(t3 * t3, axis=0, keepdims=True)

        @pl.when(i == 0)
        def _():
            s1_ref[...] = rs
            s2_ref[...] = rq

        @pl.when(i > 0)
        def _():
            s1_ref[...] = s1_ref[...] + rs
            s2_ref[...] = s2_ref[...] + rq

    @pl.when(i >= B)
    def _normalize():
        bb = i - B
        inv_n = 1.0 / BN_COUNT
        mean = s1_ref[...] * inv_n
        var = s2_ref[...] * inv_n - mean * mean
        scale = g_ref[...] * jax.lax.rsqrt(var + EPS)
        shift = be_ref[...] - mean * scale
        out_ref[0] = t3_ref[bb] * scale + shift


@functools.partial(jax.jit, static_argnames=())
def kernel(X, A_hat, t1_w1, t1_b1, t1_w2, t1_b2, t1_w3, t1_b3, Theta1,
           t2_w1, t2_b1, t2_w2, t2_b2, t2_w3, t2_b3, bn_gamma, bn_beta):
    bf = jnp.bfloat16
    w13 = t1_w1 + t1_w3
    band13 = jnp.concatenate([w13[:, :, 0, k] for k in range(3)], 1).astype(bf)
    band2 = jnp.concatenate([t1_w2[:, :, 0, k] for k in range(3)], 1).astype(bf)
    thsm = Theta1.T.astype(bf)
    v13 = t2_w1 + t2_w3
    vband13 = jnp.concatenate([v13[:, :, 0, k] for k in range(3)], 1).astype(bf)
    vband2 = jnp.concatenate([t2_w2[:, :, 0, k] for k in range(3)], 1).astype(bf)
    b13 = jnp.tile(t1_b1 + t1_b3, T1)[:, None]
    b2 = jnp.tile(t1_b2, T1)[:, None]
    c13 = jnp.tile(t2_b1 + t2_b3, T2)[:, None]
    c2 = jnp.tile(t2_b2, T2)[:, None]
    xt = jnp.transpose(X, (0, 2, 3, 1)).reshape(B, T * C_IN, N)
    g = bn_gamma[None, :]
    be = bn_beta[None, :]

    full = lambda shape: pl.BlockSpec(shape, lambda i: (0,) * len(shape))
    out = pl.pallas_call(
        _stgcn_body,
        grid=(2 * B,),
        in_specs=[
            pl.BlockSpec((1, T * C_IN, N),
                         lambda i: (jnp.minimum(i, B - 1), 0, 0)),
            full((N, N)),
            full((C_OUT, 3 * C_IN)),
            full((C_OUT, 3 * C_IN)),
            full((C_SP, C_OUT)),
            full((C_OUT, 3 * C_SP)),
            full((C_OUT, 3 * C_SP)),
            full((H1, 1)),
            full((H1, 1)),
            full((H2, 1)),
            full((H2, 1)),
            full((1, N)),
            full((1, N)),
        ],
        out_specs=pl.BlockSpec((1, H2, N),
                               lambda i: (jnp.maximum(i - B, 0), 0, 0)),
        out_shape=jax.ShapeDtypeStruct((B, H2, N), jnp.float32),
        scratch_shapes=[
            pltpu.VMEM((B, H2, N), jnp.float32),
            pltpu.VMEM((1, N), jnp.float32),
            pltpu.VMEM((1, N), jnp.float32),
            pltpu.VMEM((N, N), jnp.bfloat16),
            pltpu.VMEM((2 * H1, T * C_IN), jnp.bfloat16),
            pltpu.VMEM((T1 * C_SP, H1), jnp.bfloat16),
            pltpu.VMEM((2 * H2, T1 * C_SP), jnp.bfloat16),
        ],
    )(xt, A_hat, band13, band2, thsm, vband13, vband2,
      b13, b2, c13, c2, g, be)
    return jnp.transpose(out.reshape(B, T2, C_OUT, N), (0, 3, 1, 2))


# speedup vs baseline: 1.9158x; 1.0068x over previous
import functools

import jax
import jax.numpy as jnp
from jax.experimental import pallas as pl
from jax.experimental.pallas import tpu as pltpu

B, N, T, C_IN, C_SP, C_OUT = 8, 1024, 16, 32, 16, 32
T1 = T - 2
T2 = T1 - 2
H1 = T1 * C_OUT
H2 = T2 * C_OUT
BN_COUNT = B * T2 * C_OUT
EPS = 1e-5


def _stgcn_body(x_ref, a_ref, band13_ref, band2_ref, thsm_ref,
                vband13_ref, vband2_ref, b13_ref, b2_ref, c13_ref, c2_ref,
                g_ref, be_ref, out_ref,
                t3_ref, s1_ref, s2_ref, wc_s, th_s, vc_s):
    i = pl.program_id(0)

    @pl.when(i == 0)
    def _build_weights():
        wc_s[...] = jnp.zeros((2 * H1, T * C_IN), jnp.bfloat16)
        th_s[...] = jnp.zeros((T1 * C_SP, H1), jnp.bfloat16)
        vc_s[...] = jnp.zeros((2 * H2, T1 * C_SP), jnp.bfloat16)
        for p in range(T1):
            wc_s[32 * p:32 * p + 32, 32 * p:32 * p + 96] = band13_ref[...]
            wc_s[H1 + 32 * p:H1 + 32 * p + 32, 32 * p:32 * p + 96] = band2_ref[...]
            th_s[16 * p:16 * p + 16, 32 * p:32 * p + 32] = thsm_ref[...]
        for p in range(T2):
            vc_s[32 * p:32 * p + 32, 16 * p:16 * p + 48] = vband13_ref[...]
            vc_s[H2 + 32 * p:H2 + 32 * p + 32, 16 * p:16 * p + 48] = vband2_ref[...]

    @pl.when(i < B)
    def _compute():
        x = x_ref[0].astype(jnp.bfloat16)

        zc = jnp.dot(wc_s[...], x, preferred_element_type=jnp.float32)
        z13 = zc[:H1] + b13_ref[...]
        z2 = zc[H1:] + b2_ref[...]
        sig = 1.0 / (1.0 + jnp.exp(-z2))
        t_feat = jnp.maximum(z13 + sig, 0.0).astype(jnp.bfloat16)

        u = jnp.dot(th_s[...], t_feat, preferred_element_type=jnp.float32)
        m = jax.lax.dot_general(u, a_ref[...],
                                (((1,), (1,)), ((), ())),
                                preferred_element_type=jnp.float32)
        t2 = jnp.maximum(m, 0.0).astype(jnp.bfloat16)

        yc = jnp.dot(vc_s[...], t2, preferred_element_type=jnp.float32)
        y13 = yc[:H2] + c13_ref[...]
        y2 = yc[H2:] + c2_ref[...]
        sig2 = 1.0 / (1.0 + jnp.exp(-y2))
        t3 = jnp.maximum(y13 + sig2, 0.0)

        t3_ref[i] = t3
        rs = jnp.sum(t3, axis=0, keepdims=True)
        rq = jnp.sum(t3 * t3, axis=0, keepdims=True)

        @pl.when(i == 0)
        def _():
            s1_ref[...] = rs
            s2_ref[...] = rq

        @pl.when(i > 0)
        def _():
            s1_ref[...] = s1_ref[...] + rs
            s2_ref[...] = s2_ref[...] + rq

    @pl.when(i >= B)
    def _normalize():
        bb = i - B
        inv_n = 1.0 / BN_COUNT
        mean = s1_ref[...] * inv_n
        var = s2_ref[...] * inv_n - mean * mean
        scale = g_ref[...] * jax.lax.rsqrt(var + EPS)
        shift = be_ref[...] - mean * scale
        out_ref[0] = t3_ref[bb] * scale + shift


@functools.partial(jax.jit, static_argnames=())
def kernel(X, A_hat, t1_w1, t1_b1, t1_w2, t1_b2, t1_w3, t1_b3, Theta1,
           t2_w1, t2_b1, t2_w2, t2_b2, t2_w3, t2_b3, bn_gamma, bn_beta):
    bf = jnp.bfloat16
    w13 = t1_w1 + t1_w3
    band13 = jnp.concatenate([w13[:, :, 0, k] for k in range(3)], 1).astype(bf)
    band2 = jnp.concatenate([t1_w2[:, :, 0, k] for k in range(3)], 1).astype(bf)
    thsm = Theta1.T.astype(bf)
    v13 = t2_w1 + t2_w3
    vband13 = jnp.concatenate([v13[:, :, 0, k] for k in range(3)], 1).astype(bf)
    vband2 = jnp.concatenate([t2_w2[:, :, 0, k] for k in range(3)], 1).astype(bf)
    b13 = jnp.tile(t1_b1 + t1_b3, T1)[:, None]
    b2 = jnp.tile(t1_b2, T1)[:, None]
    c13 = jnp.tile(t2_b1 + t2_b3, T2)[:, None]
    c2 = jnp.tile(t2_b2, T2)[:, None]
    xt = jnp.transpose(X, (0, 2, 3, 1)).reshape(B, T * C_IN, N)
    g = bn_gamma[None, :]
    be = bn_beta[None, :]

    full = lambda shape: pl.BlockSpec(shape, lambda i: (0,) * len(shape))
    out = pl.pallas_call(
        _stgcn_body,
        grid=(2 * B,),
        in_specs=[
            pl.BlockSpec((1, T * C_IN, N),
                         lambda i: (jnp.minimum(i, B - 1), 0, 0)),
            full((N, N)),
            full((C_OUT, 3 * C_IN)),
            full((C_OUT, 3 * C_IN)),
            full((C_SP, C_OUT)),
            full((C_OUT, 3 * C_SP)),
            full((C_OUT, 3 * C_SP)),
            full((H1, 1)),
            full((H1, 1)),
            full((H2, 1)),
            full((H2, 1)),
            full((1, N)),
            full((1, N)),
        ],
        out_specs=pl.BlockSpec((1, H2, N),
                               lambda i: (jnp.maximum(i - B, 0), 0, 0)),
        out_shape=jax.ShapeDtypeStruct((B, H2, N), jnp.float32),
        scratch_shapes=[
            pltpu.VMEM((B, H2, N), jnp.float32),
            pltpu.VMEM((1, N), jnp.float32),
            pltpu.VMEM((1, N), jnp.float32),
            pltpu.VMEM((2 * H1, T * C_IN), jnp.bfloat16),
            pltpu.VMEM((T1 * C_SP, H1), jnp.bfloat16),
            pltpu.VMEM((2 * H2, T1 * C_SP), jnp.bfloat16),
        ],
    )(xt, A_hat, band13, band2, thsm, vband13, vband2,
      b13, b2, c13, c2, g, be)
    return jnp.transpose(out.reshape(B, T2, C_OUT, N), (0, 3, 1, 2))
